# Initial kernel scaffold; baseline (speedup 1.0000x reference)
import jax
import jax.numpy as jnp
from jax.experimental import pallas as pl


def kernel(x, centers):
    raise NotImplementedError("write your pallas kernel here")


# speedup vs baseline: 2.5481x; 2.5481x over previous
import jax
import jax.numpy as jnp
from jax.experimental import pallas as pl
from jax.experimental.pallas import tpu as pltpu

_EPS = 1e-12
_M = 1.7
_PHALF = -1.0 / (_M - 1.0)


def _fcm_body(x_ref, c_ref, c2_ref, o_ref):
    x = x_ref[...]
    c = c_ref[...]
    x2 = jnp.sum(x * x, axis=1, keepdims=True)
    xc = jax.lax.dot_general(
        x, c, (((1,), (1,)), ((), ())),
        preferred_element_type=jnp.float32)
    sq = x2 + c2_ref[...] - 2.0 * xc
    t = jnp.maximum(sq, _EPS * _EPS)
    u = jnp.exp(_PHALF * jnp.log(t))
    o_ref[...] = u / jnp.sum(u, axis=1, keepdims=True)


def kernel(x, centers):
    N, F = x.shape
    K = centers.shape[0]
    BN = 1024
    c2 = jnp.sum(centers * centers, axis=1)[None, :]
    return pl.pallas_call(
        _fcm_body,
        grid=(N // BN,),
        in_specs=[
            pl.BlockSpec((BN, F), lambda i: (i, 0)),
            pl.BlockSpec((K, F), lambda i: (0, 0)),
            pl.BlockSpec((1, K), lambda i: (0, 0)),
        ],
        out_specs=pl.BlockSpec((BN, K), lambda i: (i, 0)),
        out_shape=jax.ShapeDtypeStruct((N, K), jnp.float32),
        compiler_params=pltpu.CompilerParams(
            dimension_semantics=("parallel",)),
    )(x, centers, c2)


# speedup vs baseline: 3.4259x; 1.3445x over previous
import jax
import jax.numpy as jnp
from jax.experimental import pallas as pl
from jax.experimental.pallas import tpu as pltpu

_EPS = 1e-12
_M = 1.7
_PHALF = -1.0 / (_M - 1.0)


def _fcm_body(x_ref, c_ref, c2_ref, o_ref):
    x = x_ref[...]
    c = c_ref[...]
    x2 = jnp.sum(x * x, axis=1, keepdims=True)
    xc = jax.lax.dot_general(
        x, c, (((1,), (1,)), ((), ())),
        preferred_element_type=jnp.float32)
    sq = x2 + c2_ref[...] - 2.0 * xc
    t = jnp.maximum(sq, _EPS * _EPS)
    u = jnp.exp(_PHALF * jnp.log(t))
    o_ref[...] = u / jnp.sum(u, axis=1, keepdims=True)


def kernel(x, centers):
    N, F = x.shape
    K = centers.shape[0]
    BN = 2048
    c2 = jnp.sum(centers * centers, axis=1)[None, :]
    return pl.pallas_call(
        _fcm_body,
        grid=(N // BN,),
        in_specs=[
            pl.BlockSpec((BN, F), lambda i: (i, 0)),
            pl.BlockSpec((K, F), lambda i: (0, 0)),
            pl.BlockSpec((1, K), lambda i: (0, 0)),
        ],
        out_specs=pl.BlockSpec((BN, K), lambda i: (i, 0)),
        out_shape=jax.ShapeDtypeStruct((N, K), jnp.float32),
        compiler_params=pltpu.CompilerParams(
            dimension_semantics=("parallel",)),
    )(x, centers, c2)


# speedup vs baseline: 4.2412x; 1.2380x over previous
import jax
import jax.numpy as jnp
from jax.experimental import pallas as pl
from jax.experimental.pallas import tpu as pltpu

_EPS = 1e-12
_M = 1.7
_PHALF = -1.0 / (_M - 1.0)


def _fcm_body(x_ref, c_ref, c2_ref, o_ref):
    x = x_ref[...]
    c = c_ref[...]
    x2 = jnp.sum(x * x, axis=1, keepdims=True)
    xc = jax.lax.dot_general(
        x, c, (((1,), (1,)), ((), ())),
        preferred_element_type=jnp.float32)
    sq = x2 + c2_ref[...] - 2.0 * xc
    t = jnp.maximum(sq, _EPS * _EPS)
    u = jnp.exp(_PHALF * jnp.log(t))
    o_ref[...] = u / jnp.sum(u, axis=1, keepdims=True)


def kernel(x, centers):
    N, F = x.shape
    K = centers.shape[0]
    BN = 4096
    c2 = jnp.sum(centers * centers, axis=1)[None, :]
    return pl.pallas_call(
        _fcm_body,
        grid=(N // BN,),
        in_specs=[
            pl.BlockSpec((BN, F), lambda i: (i, 0)),
            pl.BlockSpec((K, F), lambda i: (0, 0)),
            pl.BlockSpec((1, K), lambda i: (0, 0)),
        ],
        out_specs=pl.BlockSpec((BN, K), lambda i: (i, 0)),
        out_shape=jax.ShapeDtypeStruct((N, K), jnp.float32),
        compiler_params=pltpu.CompilerParams(
            dimension_semantics=("parallel",)),
    )(x, centers, c2)


# speedup vs baseline: 4.8099x; 1.1341x over previous
import jax
import jax.numpy as jnp
from jax.experimental import pallas as pl
from jax.experimental.pallas import tpu as pltpu

_EPS = 1e-12
_M = 1.7
_PHALF = -1.0 / (_M - 1.0)


def _fcm_body(x_ref, c_ref, c2_ref, o_ref):
    x = x_ref[...]
    c = c_ref[...]
    x2 = jnp.sum(x * x, axis=1, keepdims=True)
    xc = jax.lax.dot_general(
        x, c, (((1,), (1,)), ((), ())),
        preferred_element_type=jnp.float32)
    sq = x2 + c2_ref[...] - 2.0 * xc
    t = jnp.maximum(sq, _EPS * _EPS)
    u = jnp.exp(_PHALF * jnp.log(t))
    o_ref[...] = u / jnp.sum(u, axis=1, keepdims=True)


def kernel(x, centers):
    N, F = x.shape
    K = centers.shape[0]
    BN = 8192
    c2 = jnp.sum(centers * centers, axis=1)[None, :]
    return pl.pallas_call(
        _fcm_body,
        grid=(N // BN,),
        in_specs=[
            pl.BlockSpec((BN, F), lambda i: (i, 0)),
            pl.BlockSpec((K, F), lambda i: (0, 0)),
            pl.BlockSpec((1, K), lambda i: (0, 0)),
        ],
        out_specs=pl.BlockSpec((BN, K), lambda i: (i, 0)),
        out_shape=jax.ShapeDtypeStruct((N, K), jnp.float32),
        compiler_params=pltpu.CompilerParams(
            dimension_semantics=("parallel",)),
    )(x, centers, c2)
